# baseline (speedup 1.0000x reference)
import functools

import jax
import jax.numpy as jnp
from jax import lax
from jax.experimental import pallas as pl
from jax.experimental.pallas import tpu as pltpu
from jax.experimental.pallas import tpu_sc as plsc

N = 10000
E = 320000
D = 128
H1 = 4
HID = 128
DOC = 256
B = 16
OUT = 10

NC = 2
NS = 16
NW = NC * NS

NP = 10240
E_REAL = E + N
CH = 64
E_PAD = 331776
EPW = E_PAD // NW


def _mesh():
    return plsc.VectorSubcoreMesh(core_axis_name="c", subcore_axis_name="s")


_SC_PARAMS = pltpu.CompilerParams(needs_layout_passes=False)


def _make_pass1(H):
    ROW = H * HID
    HNP = H * NP
    C1 = 32 if H > 1 else 64
    NCHUNK = EPW // C1
    SB = 12 if H > 1 else 6
    PB = SB

    def body(xl_hbm, xr_hbm, src_hbm, dst_hbm, att_hbm,
             p_hbm, den_hbm,
             srcb_v, dstb_v, a0_v, a1_v, b0_v, b1_v, att_v, acc_v, pstage_v,
             sa0, sa1, sb0, sb1):
        c = lax.axis_index("c")
        s = lax.axis_index("s")
        wid = s * NC + c
        tilebase = wid * EPW

        zero16 = jnp.zeros((16,), jnp.float32)
        lane = lax.iota(jnp.int32, 16)

        def zbody(i, carry):
            acc_v[pl.ds(i * 16, 16)] = zero16
            return carry
        lax.fori_loop(0, HNP // 16, zbody, 0)

        pltpu.sync_copy(att_hbm, att_v)
        att_regs = [[att_v[pl.ds(h * HID + k * 16, 16)]
                     for k in range(HID // 16)] for h in range(H)]

        a_refs = (a0_v, a1_v)
        b_refs = (b0_v, b1_v)
        sa_refs = (sa0, sa1)
        sb_refs = (sb0, sb1)

        def load_batch(bi):
            bslot = lax.rem(bi, 2)
            pltpu.sync_copy(src_hbm.at[pl.ds(tilebase + bi * SB * C1,
                                             SB * C1)], srcb_v.at[bslot])
            pltpu.sync_copy(dst_hbm.at[pl.ds(tilebase + bi * SB * C1,
                                             SB * C1)], dstb_v.at[bslot])

        def issue_gather(ci, sl):
            bslot = lax.rem(ci // SB, 2)
            off = lax.rem(ci, SB) * C1
            pltpu.async_copy(xl_hbm.at[srcb_v.at[bslot, pl.ds(off, C1)]],
                             a_refs[sl], sa_refs[sl])
            pltpu.async_copy(xr_hbm.at[dstb_v.at[bslot, pl.ds(off, C1)]],
                             b_refs[sl], sb_refs[sl])

        def wait_gather(ci, sl):
            bslot = lax.rem(ci // SB, 2)
            off = lax.rem(ci, SB) * C1
            pltpu.make_async_copy(xl_hbm.at[srcb_v.at[bslot, pl.ds(off, C1)]],
                                  a_refs[sl], sa_refs[sl]).wait()
            pltpu.make_async_copy(xr_hbm.at[dstb_v.at[bslot, pl.ds(off, C1)]],
                                  b_refs[sl], sb_refs[sl]).wait()

        def compute_chunk(ci, sl):
            a_v = a_refs[sl]
            b_v = b_refs[sl]
            bslot = lax.rem(ci // SB, 2)
            coff = lax.rem(ci, SB) * C1
            poff = lax.rem(ci, PB) * C1
            base = tilebase + ci * C1
            for g in range(C1 // 16):
                def edge_body(e, lvecs):
                    row = g * 16 + e
                    out = []
                    for h in range(H):
                        acc = zero16
                        for k in range(HID // 16):
                            col = h * HID + k * 16
                            av = a_v[row, pl.ds(col, 16)]
                            bv = b_v[row, pl.ds(col, 16)]
                            z = av + bv
                            lr = jnp.maximum(z, 0.2 * z)
                            acc = acc + lr * att_regs[h][k]
                        sval = jnp.sum(acc)
                        out.append(jnp.where(lane == e, sval, lvecs[h]))
                    return tuple(out)
                lvecs = lax.fori_loop(0, 16, edge_body, (zero16,) * H)
                mask = (base + g * 16 + lane) < E_REAL
                dvec = dstb_v[bslot, pl.ds(coff + g * 16, 16)]
                pidx = (poff + g * 16 + lane) * H
                for h in range(H):
                    pv = jnp.exp(lvecs[h])
                    pv = jnp.where(mask, pv, 0.0)
                    plsc.store_scatter(pstage_v, [pidx + h], pv)
                    idx = dvec + h * NP
                    for e in range(16):
                        plsc.addupdate_scatter(acc_v, [idx], pv,
                                               mask=lane == e)

        load_batch(0)
        issue_gather(0, 0)
        issue_gather(1, 1)

        def pair_body(ci2, carry):
            for sl in (0, 1):
                ci = ci2 * 2 + sl
                wait_gather(ci, sl)
                compute_chunk(ci, sl)
                nci = ci + 2

                @pl.when(jnp.logical_and(lax.rem(nci, SB) == 0,
                                         nci < NCHUNK))
                def _():
                    load_batch(nci // SB)

                @pl.when(nci < NCHUNK)
                def _():
                    issue_gather(nci, sl)

                @pl.when(lax.rem(ci + 1, PB) == 0)
                def _():
                    pltpu.sync_copy(
                        pstage_v,
                        p_hbm.at[pl.ds((tilebase + (ci + 1 - PB) * C1) * H,
                                       PB * C1 * H)])
            return carry
        lax.fori_loop(0, NCHUNK // 2, pair_body, 0)
        pltpu.sync_copy(acc_v, den_hbm.at[wid])

    return pl.kernel(
        body,
        compiler_params=_SC_PARAMS,
        out_type=(
            jax.ShapeDtypeStruct((E_PAD * H,), jnp.float32),
            jax.ShapeDtypeStruct((NW, HNP), jnp.float32),
        ),
        mesh=_mesh(),
        scratch_types=(
            pltpu.VMEM((2, SB * C1), jnp.int32),
            pltpu.VMEM((2, SB * C1), jnp.int32),
            pltpu.VMEM((C1, ROW), jnp.float32),
            pltpu.VMEM((C1, ROW), jnp.float32),
            pltpu.VMEM((C1, ROW), jnp.float32),
            pltpu.VMEM((C1, ROW), jnp.float32),
            pltpu.VMEM((ROW,), jnp.float32),
            pltpu.VMEM((HNP,), jnp.float32),
            pltpu.VMEM((PB * C1 * H,), jnp.float32),
            pltpu.SemaphoreType.DMA,
            pltpu.SemaphoreType.DMA,
            pltpu.SemaphoreType.DMA,
            pltpu.SemaphoreType.DMA,
        ),
    )


def _make_pass2(H):
    NPLANE = H if H > 1 else NC
    if H > 1:
        EPT = E_PAD // NS
        HEADS_PER_SC = H // NC
        SB = 12
    else:
        EPT = E_PAD // NW
        HEADS_PER_SC = 1
        SB = 6
    C2 = 64
    NCHUNK = EPT // C2
    NROWS = NP // NS
    WB = 32

    def body(xlv_hbm, src_hbm, dst_hbm, p_hbm, den_hbm,
             out_hbm,
             srcb_v, dstb_v, pb_v, i0_v, i1_v, i2_v, d0_v, d1_v, d2_v,
             r0_v, r1_v, r2_v, den_v, wout_v, q_sh,
             sg0, sg1, sg2, ss0, ss1, ss2):
        c = lax.axis_index("c")
        s = lax.axis_index("s")

        zero16 = jnp.zeros((16,), jnp.float32)

        i_refs = (i0_v, i1_v, i2_v)
        d_refs = (d0_v, d1_v, d2_v)
        r_refs = (r0_v, r1_v, r2_v)
        sg_refs = (sg0, sg1, sg2)
        ss_refs = (ss0, ss1, ss2)

        def zero_wout():
            def zb(r, carry):
                for k in range(HID // 16):
                    wout_v[r, pl.ds(k * 16, 16)] = zero16
                return carry
            lax.fori_loop(0, WB, zb, 0)

        def zero_q():
            zero_wout()
            for j in range(NROWS // WB):
                pltpu.sync_copy(wout_v, q_sh.at[pl.ds(s * NROWS + j * WB, WB)])

        def run_task(t):
            if H > 1:
                h = c * HEADS_PER_SC + t
                tbase = s * EPT
                plane = h
            else:
                h = 0
                tbase = (c * NS + s) * EPT
                plane = c
            pltpu.sync_copy(den_hbm.at[pl.ds(h * NP, NP)], den_v)

            def load_batch(bi):
                bslot = lax.rem(bi, 2)
                eb = tbase + bi * SB * C2
                pltpu.sync_copy(src_hbm.at[pl.ds(eb, SB * C2)],
                                srcb_v.at[bslot])
                pltpu.sync_copy(dst_hbm.at[pl.ds(eb, SB * C2)],
                                dstb_v.at[bslot])
                pltpu.sync_copy(p_hbm.at[pl.ds(eb * H, SB * C2 * H)],
                                pb_v.at[pl.ds(bslot * SB * C2 * H,
                                              SB * C2 * H)])

            def prep_idx(ci, sl):
                bslot = lax.rem(ci // SB, 2)
                coff = lax.rem(ci, SB) * C2
                for g in range(C2 // 16):
                    sv = srcb_v[bslot, pl.ds(coff + g * 16, 16)]
                    if H > 1:
                        i_refs[sl][pl.ds(g * 16, 16)] = sv * H + h
                    else:
                        i_refs[sl][pl.ds(g * 16, 16)] = sv

            def issue_gather(sl):
                pltpu.async_copy(xlv_hbm.at[i_refs[sl]], r_refs[sl],
                                 sg_refs[sl])

            def wait_gather(sl):
                pltpu.make_async_copy(xlv_hbm.at[i_refs[sl]], r_refs[sl],
                                      sg_refs[sl]).wait()

            def compute_chunk(ci, sl):
                bslot = lax.rem(ci // SB, 2)
                coff = lax.rem(ci, SB) * C2
                rows_v = r_refs[sl]
                for g in range(C2 // 16):
                    glane = lax.iota(jnp.int32, 16)
                    pidx = (bslot * SB * C2 + coff + g * 16 + glane) * H + h
                    pv = plsc.load_gather(pb_v, [pidx])
                    dv = dstb_v[bslot, pl.ds(coff + g * 16, 16)]
                    dg = plsc.load_gather(den_v, [dv])
                    al = pv / (dg + 1e-16)
                    d_refs[sl][pl.ds(g * 16, 16)] = dv
                    for e in range(16):
                        row = g * 16 + e
                        a_s = al[e]
                        for k in range(HID // 16):
                            rv = rows_v[row, pl.ds(k * 16, 16)]
                            rows_v[row, pl.ds(k * 16, 16)] = rv * a_s

            def issue_scatter(sl):
                pltpu.async_copy(r_refs[sl], q_sh.at[d_refs[sl]],
                                 ss_refs[sl], add=True)

            def wait_scatter(sl):
                pltpu.make_async_copy(r_refs[sl], q_sh.at[d_refs[sl]],
                                      ss_refs[sl]).wait()

            load_batch(0)
            for j in range(2):
                prep_idx(j, j)
                issue_gather(j)

            def tri_body(ci3, carry):
                for sl in (0, 1, 2):
                    ci = ci3 * 3 + sl
                    wait_gather(sl)
                    compute_chunk(ci, sl)
                    issue_scatter(sl)
                    nci = ci + 2
                    sl2 = (sl + 2) % 3

                    @pl.when(jnp.logical_and(lax.rem(nci, SB) == 0,
                                             nci < NCHUNK))
                    def _():
                        load_batch(nci // SB)

                    @pl.when(ci >= 1)
                    def _():
                        wait_scatter(sl2)

                    @pl.when(nci < NCHUNK)
                    def _():
                        prep_idx(nci, sl2)
                        issue_gather(sl2)
                return carry
            lax.fori_loop(0, NCHUNK // 3, tri_body, 0)
            wait_scatter((NCHUNK - 1) % 3)
            plsc.subcore_barrier()
            for j in range(NROWS // WB):
                rs = s * NROWS + j * WB
                pltpu.sync_copy(q_sh.at[pl.ds(rs, WB)], wout_v)
                pltpu.sync_copy(wout_v, out_hbm.at[plane, pl.ds(rs, WB)])
            plsc.subcore_barrier()

        zero_q()
        plsc.subcore_barrier()
        for t in range(HEADS_PER_SC):
            if t > 0:
                zero_q()
                plsc.subcore_barrier()
            run_task(t)

    return pl.kernel(
        body,
        compiler_params=_SC_PARAMS,
        out_type=jax.ShapeDtypeStruct((NPLANE, NP, HID), jnp.float32),
        mesh=_mesh(),
        scratch_types=(
            pltpu.VMEM((2, SB * C2), jnp.int32),
            pltpu.VMEM((2, SB * C2), jnp.int32),
            pltpu.VMEM((2 * SB * C2 * H,), jnp.float32),
            pltpu.VMEM((C2,), jnp.int32),
            pltpu.VMEM((C2,), jnp.int32),
            pltpu.VMEM((C2,), jnp.int32),
            pltpu.VMEM((C2,), jnp.int32),
            pltpu.VMEM((C2,), jnp.int32),
            pltpu.VMEM((C2,), jnp.int32),
            pltpu.VMEM((C2, HID), jnp.float32),
            pltpu.VMEM((C2, HID), jnp.float32),
            pltpu.VMEM((C2, HID), jnp.float32),
            pltpu.VMEM((NP,), jnp.float32),
            pltpu.VMEM((WB, HID), jnp.float32),
            pltpu.VMEM_SHARED((NP, HID), jnp.float32),
            pltpu.SemaphoreType.DMA,
            pltpu.SemaphoreType.DMA,
            pltpu.SemaphoreType.DMA,
            pltpu.SemaphoreType.DMA,
            pltpu.SemaphoreType.DMA,
            pltpu.SemaphoreType.DMA,
        ),
    )


_pass1_l1 = _make_pass1(H1)
_pass1_l2 = _make_pass1(1)
_pass2_l1 = _make_pass2(H1)
_pass2_l2 = _make_pass2(1)


def _proj1_body(x_ref, wl_ref, wr_ref, ol_ref, or_ref):
    xb = x_ref[...]
    ol_ref[...] = xb @ wl_ref[...]
    or_ref[...] = xb @ wr_ref[...]


def _proj1(x, Wl, Wr):
    bn = 1024
    return pl.pallas_call(
        _proj1_body,
        grid=(NP // bn,),
        in_specs=[
            pl.BlockSpec((bn, D), lambda i: (i, 0)),
            pl.BlockSpec((D, H1 * HID), lambda i: (0, 0)),
            pl.BlockSpec((D, H1 * HID), lambda i: (0, 0)),
        ],
        out_specs=(
            pl.BlockSpec((bn, H1 * HID), lambda i: (i, 0)),
            pl.BlockSpec((bn, H1 * HID), lambda i: (i, 0)),
        ),
        out_shape=(
            jax.ShapeDtypeStruct((NP, H1 * HID), jnp.float32),
            jax.ShapeDtypeStruct((NP, H1 * HID), jnp.float32),
        ),
    )(x, Wl, Wr)


def _proj2_body(q_ref, b_ref, wl_ref, wr_ref, ol_ref, or_ref):
    hcat = jnp.concatenate([q_ref[j] for j in range(H1)], axis=-1)
    hb = jnp.maximum(hcat + b_ref[...], 0.0)
    ol_ref[...] = hb @ wl_ref[...]
    or_ref[...] = hb @ wr_ref[...]


def _proj2(q1, b1, Wl2, Wr2):
    bn = 1024
    return pl.pallas_call(
        _proj2_body,
        grid=(NP // bn,),
        in_specs=[
            pl.BlockSpec((H1, bn, HID), lambda i: (0, i, 0)),
            pl.BlockSpec((1, H1 * HID), lambda i: (0, 0)),
            pl.BlockSpec((H1 * HID, HID), lambda i: (0, 0)),
            pl.BlockSpec((H1 * HID, HID), lambda i: (0, 0)),
        ],
        out_specs=(
            pl.BlockSpec((bn, HID), lambda i: (i, 0)),
            pl.BlockSpec((bn, HID), lambda i: (i, 0)),
        ),
        out_shape=(
            jax.ShapeDtypeStruct((NP, HID), jnp.float32),
            jax.ShapeDtypeStruct((NP, HID), jnp.float32),
        ),
    )(q1, b1.reshape(1, -1), Wl2, Wr2)


def _redsum_body(d_ref, o_ref):
    o_ref[...] = jnp.sum(d_ref[...], axis=0, keepdims=True)


def _reduce_den(den):
    hnp = den.shape[1]
    bn = 2048
    out = pl.pallas_call(
        _redsum_body,
        grid=(hnp // bn,),
        in_specs=[pl.BlockSpec((NW, bn), lambda i: (0, i))],
        out_specs=pl.BlockSpec((1, bn), lambda i: (0, i)),
        out_shape=jax.ShapeDtypeStruct((1, hnp), jnp.float32),
    )(den)
    return out.reshape(hnp)


def _tail_body(q_ref, b2_ref, batch_ref, doc_ref, Wdoc_ref, bdoc_ref,
               gamma_ref, beta_ref, Wfus_ref, bfus_ref, Wtask_ref, btask_ref,
               Wtime_ref, btime_ref, task_ref, time_ref):
    nblk = NP // 128
    iot = lax.broadcasted_iota(jnp.int32, (B, 128), 0)

    def blk(i, carry):
        pooled, cnt = carry
        hb = q_ref[0, pl.ds(i * 128, 128), :] + q_ref[1, pl.ds(i * 128, 128), :]
        hb = jnp.maximum(hb + b2_ref[...], 0.0)
        bv = batch_ref[i, :]
        onehot = (bv[None, :] == iot).astype(jnp.float32)
        pooled = pooled + onehot @ hb
        cnt = cnt + jnp.sum(onehot, axis=1, keepdims=True)
        return (pooled, cnt)

    pooled, cnt = lax.fori_loop(
        0, nblk, blk,
        (jnp.zeros((B, HID), jnp.float32), jnp.zeros((B, 1), jnp.float32)))
    pooled = pooled / jnp.maximum(cnt, 1.0)
    doc = jnp.maximum(doc_ref[...] @ Wdoc_ref[...] + bdoc_ref[...], 0.0)
    fusion = jnp.concatenate([pooled, doc], axis=1)
    mu = jnp.mean(fusion, axis=0, keepdims=True)
    var = jnp.mean((fusion - mu) ** 2, axis=0, keepdims=True)
    fusion = (fusion - mu) / jnp.sqrt(var + 1e-5) * gamma_ref[...] + beta_ref[...]
    fusion = jnp.maximum(fusion @ Wfus_ref[...] + bfus_ref[...], 0.0)
    task_ref[...] = fusion @ Wtask_ref[...] + btask_ref[...]
    time_ref[...] = fusion @ Wtime_ref[...] + btime_ref[...]


def _tail(q2, b2, batch2d, doc, Wdoc, bdoc, gamma, beta, Wfus, bfus,
          Wtask, btask, Wtime, btime):
    return pl.pallas_call(
        _tail_body,
        out_shape=(
            jax.ShapeDtypeStruct((B, OUT), jnp.float32),
            jax.ShapeDtypeStruct((B, 1), jnp.float32),
        ),
    )(q2, b2.reshape(1, -1), batch2d, doc, Wdoc, bdoc.reshape(1, -1),
      gamma.reshape(1, -1), beta.reshape(1, -1), Wfus, bfus.reshape(1, -1),
      Wtask, btask.reshape(1, -1), Wtime, btime.reshape(1, -1))


def kernel(x, edge_index, batch, doc_features, Wl1, Wr1, att1, b1, Wl2, Wr2,
           att2, b2, Wdoc, bdoc, gamma, beta, Wfus, bfus, Wtask, btask, Wtime,
           btime):
    loop = jnp.arange(N, dtype=edge_index.dtype)
    srcp = jnp.concatenate(
        [edge_index[0], loop, jnp.zeros((E_PAD - E_REAL,), jnp.int32)])
    dstp = jnp.concatenate(
        [edge_index[1], loop, jnp.zeros((E_PAD - E_REAL,), jnp.int32)])
    xp = jnp.pad(x, ((0, NP - N), (0, 0)))
    batchp = jnp.concatenate(
        [batch, jnp.full((NP - N,), B, jnp.int32)]).reshape(NP // 128, 128)

    xl1, xr1 = _proj1(xp, Wl1, Wr1)
    p1, den1p = _pass1_l1(xl1, xr1, srcp, dstp, att1.reshape(-1))
    den1 = _reduce_den(den1p)
    q1 = _pass2_l1(xl1.reshape(NP * H1, HID), srcp, dstp, p1, den1)

    xl2, xr2 = _proj2(q1, b1, Wl2, Wr2)
    p2, den2p = _pass1_l2(xl2, xr2, srcp, dstp, att2.reshape(-1))
    den2 = _reduce_den(den2p)
    q2 = _pass2_l2(xl2, srcp, dstp, p2, den2)

    task, time = _tail(q2, b2, batchp, doc_features, Wdoc, bdoc, gamma, beta,
                       Wfus, bfus, Wtask, btask, Wtime, btime)
    return (task, time)


# speedup vs baseline: 1.0596x; 1.0596x over previous
import functools

import jax
import jax.numpy as jnp
from jax import lax
from jax.experimental import pallas as pl
from jax.experimental.pallas import tpu as pltpu
from jax.experimental.pallas import tpu_sc as plsc

N = 10000
E = 320000
D = 128
H1 = 4
HID = 128
DOC = 256
B = 16
OUT = 10

NC = 2
NS = 16
NW = NC * NS

NP = 10240
E_REAL = E + N
CH = 64
E_PAD = 331776
EPW = E_PAD // NW


def _mesh():
    return plsc.VectorSubcoreMesh(core_axis_name="c", subcore_axis_name="s")


_SC_PARAMS = pltpu.CompilerParams(needs_layout_passes=False)


def _make_pass1(H):
    ROW = H * HID
    HNP = H * NP
    C1 = 32 if H > 1 else 64
    NCHUNK = EPW // C1
    SB = 12 if H > 1 else 6
    PB = SB

    def body(xl_hbm, xr_hbm, src_hbm, dst_hbm, att_hbm,
             p_hbm, den_hbm,
             srcb_v, dstb_v, a0_v, a1_v, b0_v, b1_v, att_v, acc_v, pstage_v,
             sa0, sa1, sb0, sb1):
        c = lax.axis_index("c")
        s = lax.axis_index("s")
        wid = s * NC + c
        tilebase = wid * EPW

        zero16 = jnp.zeros((16,), jnp.float32)
        lane = lax.iota(jnp.int32, 16)

        def zbody(i, carry):
            acc_v[pl.ds(i * 16, 16)] = zero16
            return carry
        lax.fori_loop(0, HNP // 16, zbody, 0)

        pltpu.sync_copy(att_hbm, att_v)
        att_regs = [[att_v[pl.ds(h * HID + k * 16, 16)]
                     for k in range(HID // 16)] for h in range(H)]

        a_refs = (a0_v, a1_v)
        b_refs = (b0_v, b1_v)
        sa_refs = (sa0, sa1)
        sb_refs = (sb0, sb1)

        def load_batch(bi):
            bslot = lax.rem(bi, 2)
            pltpu.sync_copy(src_hbm.at[pl.ds(tilebase + bi * SB * C1,
                                             SB * C1)], srcb_v.at[bslot])
            pltpu.sync_copy(dst_hbm.at[pl.ds(tilebase + bi * SB * C1,
                                             SB * C1)], dstb_v.at[bslot])

        def issue_gather(ci, sl):
            bslot = lax.rem(ci // SB, 2)
            off = lax.rem(ci, SB) * C1
            pltpu.async_copy(xl_hbm.at[srcb_v.at[bslot, pl.ds(off, C1)]],
                             a_refs[sl], sa_refs[sl])
            pltpu.async_copy(xr_hbm.at[dstb_v.at[bslot, pl.ds(off, C1)]],
                             b_refs[sl], sb_refs[sl])

        def wait_gather(ci, sl):
            bslot = lax.rem(ci // SB, 2)
            off = lax.rem(ci, SB) * C1
            pltpu.make_async_copy(xl_hbm.at[srcb_v.at[bslot, pl.ds(off, C1)]],
                                  a_refs[sl], sa_refs[sl]).wait()
            pltpu.make_async_copy(xr_hbm.at[dstb_v.at[bslot, pl.ds(off, C1)]],
                                  b_refs[sl], sb_refs[sl]).wait()

        def compute_chunk(ci, sl):
            a_v = a_refs[sl]
            b_v = b_refs[sl]
            bslot = lax.rem(ci // SB, 2)
            coff = lax.rem(ci, SB) * C1
            poff = lax.rem(ci, PB) * C1
            base = tilebase + ci * C1
            for g in range(C1 // 16):
                def edge_body(e, lvecs):
                    row = g * 16 + e
                    out = []
                    for h in range(H):
                        acc = zero16
                        for k in range(HID // 16):
                            col = h * HID + k * 16
                            av = a_v[row, pl.ds(col, 16)]
                            bv = b_v[row, pl.ds(col, 16)]
                            z = av + bv
                            lr = jnp.maximum(z, 0.2 * z)
                            acc = acc + lr * att_regs[h][k]
                        sval = jnp.sum(acc)
                        out.append(jnp.where(lane == e, sval, lvecs[h]))
                    return tuple(out)
                lvecs = lax.fori_loop(0, 16, edge_body, (zero16,) * H)
                mask = (base + g * 16 + lane) < E_REAL
                dvec = dstb_v[bslot, pl.ds(coff + g * 16, 16)]
                pidx = (poff + g * 16 + lane) * H
                for h in range(H):
                    pv = jnp.exp(lvecs[h])
                    pv = jnp.where(mask, pv, 0.0)
                    plsc.store_scatter(pstage_v, [pidx + h], pv)
                    idx = dvec + h * NP
                    for e in range(16):
                        plsc.addupdate_scatter(acc_v, [idx], pv,
                                               mask=lane == e)

        load_batch(0)
        issue_gather(0, 0)
        issue_gather(1, 1)

        def pair_body(ci2, carry):
            for sl in (0, 1):
                ci = ci2 * 2 + sl
                wait_gather(ci, sl)
                compute_chunk(ci, sl)
                nci = ci + 2

                @pl.when(jnp.logical_and(lax.rem(nci, SB) == 0,
                                         nci < NCHUNK))
                def _():
                    load_batch(nci // SB)

                @pl.when(nci < NCHUNK)
                def _():
                    issue_gather(nci, sl)

                @pl.when(lax.rem(ci + 1, PB) == 0)
                def _():
                    pltpu.sync_copy(
                        pstage_v,
                        p_hbm.at[pl.ds((tilebase + (ci + 1 - PB) * C1) * H,
                                       PB * C1 * H)])
            return carry
        lax.fori_loop(0, NCHUNK // 2, pair_body, 0)
        pltpu.sync_copy(acc_v, den_hbm.at[wid])

    return pl.kernel(
        body,
        compiler_params=_SC_PARAMS,
        out_type=(
            jax.ShapeDtypeStruct((E_PAD * H,), jnp.float32),
            jax.ShapeDtypeStruct((NW, HNP), jnp.float32),
        ),
        mesh=_mesh(),
        scratch_types=(
            pltpu.VMEM((2, SB * C1), jnp.int32),
            pltpu.VMEM((2, SB * C1), jnp.int32),
            pltpu.VMEM((C1, ROW), jnp.float32),
            pltpu.VMEM((C1, ROW), jnp.float32),
            pltpu.VMEM((C1, ROW), jnp.float32),
            pltpu.VMEM((C1, ROW), jnp.float32),
            pltpu.VMEM((ROW,), jnp.float32),
            pltpu.VMEM((HNP,), jnp.float32),
            pltpu.VMEM((PB * C1 * H,), jnp.float32),
            pltpu.SemaphoreType.DMA,
            pltpu.SemaphoreType.DMA,
            pltpu.SemaphoreType.DMA,
            pltpu.SemaphoreType.DMA,
        ),
    )


def _make_pass2(H):
    NPLANE = H if H > 1 else NC
    if H > 1:
        EPT = E_PAD // NS
        HEADS_PER_SC = H // NC
        SB = 12
    else:
        EPT = E_PAD // NW
        HEADS_PER_SC = 1
        SB = 6
    C2 = 64
    NCHUNK = EPT // C2
    NROWS = NP // NS
    WB = 64

    def body(xlv_hbm, src_hbm, dst_hbm, p_hbm, den_hbm,
             out_hbm,
             srcb_v, dstb_v, pb_v, i0_v, i1_v, d0_v, d1_v, r0_v, r1_v,
             den_v, wout_v, q_sh, sg0, sg1, ss0, ss1):
        c = lax.axis_index("c")
        s = lax.axis_index("s")

        zero16 = jnp.zeros((16,), jnp.float32)

        i_refs = (i0_v, i1_v)
        d_refs = (d0_v, d1_v)
        r_refs = (r0_v, r1_v)
        sg_refs = (sg0, sg1)
        ss_refs = (ss0, ss1)

        def zero_wout():
            def zb(r, carry):
                for k in range(HID // 16):
                    wout_v[r, pl.ds(k * 16, 16)] = zero16
                return carry
            lax.fori_loop(0, WB, zb, 0)

        def zero_q():
            zero_wout()
            for j in range(NROWS // WB):
                pltpu.sync_copy(wout_v, q_sh.at[pl.ds(s * NROWS + j * WB, WB)])

        def run_task(t):
            if H > 1:
                h = c * HEADS_PER_SC + t
                tbase = s * EPT
                plane = h
            else:
                h = 0
                tbase = (c * NS + s) * EPT
                plane = c
            pltpu.sync_copy(den_hbm.at[pl.ds(h * NP, NP)], den_v)

            def load_batch(bi):
                bslot = lax.rem(bi, 2)
                eb = tbase + bi * SB * C2
                pltpu.sync_copy(src_hbm.at[pl.ds(eb, SB * C2)],
                                srcb_v.at[bslot])
                pltpu.sync_copy(dst_hbm.at[pl.ds(eb, SB * C2)],
                                dstb_v.at[bslot])
                pltpu.sync_copy(p_hbm.at[pl.ds(eb * H, SB * C2 * H)],
                                pb_v.at[pl.ds(bslot * SB * C2 * H,
                                              SB * C2 * H)])

            def prep_idx(ci, sl):
                bslot = lax.rem(ci // SB, 2)
                coff = lax.rem(ci, SB) * C2
                for g in range(C2 // 16):
                    sv = srcb_v[bslot, pl.ds(coff + g * 16, 16)]
                    if H > 1:
                        i_refs[sl][pl.ds(g * 16, 16)] = sv * H + h
                    else:
                        i_refs[sl][pl.ds(g * 16, 16)] = sv

            def issue_gather(sl):
                pltpu.async_copy(xlv_hbm.at[i_refs[sl]], r_refs[sl],
                                 sg_refs[sl])

            def wait_gather(sl):
                pltpu.make_async_copy(xlv_hbm.at[i_refs[sl]], r_refs[sl],
                                      sg_refs[sl]).wait()

            def compute_chunk(ci, sl):
                bslot = lax.rem(ci // SB, 2)
                coff = lax.rem(ci, SB) * C2
                rows_v = r_refs[sl]
                for g in range(C2 // 16):
                    glane = lax.iota(jnp.int32, 16)
                    pidx = (bslot * SB * C2 + coff + g * 16 + glane) * H + h
                    pv = plsc.load_gather(pb_v, [pidx])
                    dv = dstb_v[bslot, pl.ds(coff + g * 16, 16)]
                    dg = plsc.load_gather(den_v, [dv])
                    al = pv / (dg + 1e-16)
                    d_refs[sl][pl.ds(g * 16, 16)] = dv
                    for e in range(16):
                        row = g * 16 + e
                        a_s = al[e]
                        for k in range(HID // 16):
                            rv = rows_v[row, pl.ds(k * 16, 16)]
                            rows_v[row, pl.ds(k * 16, 16)] = rv * a_s

            def issue_scatter(sl):
                pltpu.async_copy(r_refs[sl], q_sh.at[d_refs[sl]],
                                 ss_refs[sl], add=True)

            def wait_scatter(sl):
                pltpu.make_async_copy(r_refs[sl], q_sh.at[d_refs[sl]],
                                      ss_refs[sl]).wait()

            load_batch(0)
            prep_idx(0, 0)
            issue_gather(0)
            prep_idx(1, 1)
            issue_gather(1)

            def pair_body(ci2, carry):
                for sl in (0, 1):
                    ci = ci2 * 2 + sl
                    wait_gather(sl)
                    compute_chunk(ci, sl)
                    issue_scatter(sl)
                    nci = ci + 2

                    @pl.when(jnp.logical_and(lax.rem(nci, SB) == 0,
                                             nci < NCHUNK))
                    def _():
                        load_batch(nci // SB)

                    wait_scatter(sl)

                    @pl.when(nci < NCHUNK)
                    def _():
                        prep_idx(nci, sl)
                        issue_gather(sl)
                return carry
            lax.fori_loop(0, NCHUNK // 2, pair_body, 0)
            plsc.subcore_barrier()
            for j in range(NROWS // WB):
                rs = s * NROWS + j * WB
                pltpu.sync_copy(q_sh.at[pl.ds(rs, WB)], wout_v)
                pltpu.sync_copy(wout_v, out_hbm.at[plane, pl.ds(rs, WB)])
            plsc.subcore_barrier()

        zero_q()
        plsc.subcore_barrier()
        for t in range(HEADS_PER_SC):
            if t > 0:
                zero_q()
                plsc.subcore_barrier()
            run_task(t)

    return pl.kernel(
        body,
        compiler_params=_SC_PARAMS,
        out_type=jax.ShapeDtypeStruct((NPLANE, NP, HID), jnp.float32),
        mesh=_mesh(),
        scratch_types=(
            pltpu.VMEM((2, SB * C2), jnp.int32),
            pltpu.VMEM((2, SB * C2), jnp.int32),
            pltpu.VMEM((2 * SB * C2 * H,), jnp.float32),
            pltpu.VMEM((C2,), jnp.int32),
            pltpu.VMEM((C2,), jnp.int32),
            pltpu.VMEM((C2,), jnp.int32),
            pltpu.VMEM((C2,), jnp.int32),
            pltpu.VMEM((C2, HID), jnp.float32),
            pltpu.VMEM((C2, HID), jnp.float32),
            pltpu.VMEM((NP,), jnp.float32),
            pltpu.VMEM((WB, HID), jnp.float32),
            pltpu.VMEM_SHARED((NP, HID), jnp.float32),
            pltpu.SemaphoreType.DMA,
            pltpu.SemaphoreType.DMA,
            pltpu.SemaphoreType.DMA,
            pltpu.SemaphoreType.DMA,
        ),
    )


_pass1_l1 = _make_pass1(H1)
_pass1_l2 = _make_pass1(1)
_pass2_l1 = _make_pass2(H1)
_pass2_l2 = _make_pass2(1)


def _proj1_body(x_ref, wl_ref, wr_ref, ol_ref, or_ref):
    xb = x_ref[...]
    ol_ref[...] = xb @ wl_ref[...]
    or_ref[...] = xb @ wr_ref[...]


def _proj1(x, Wl, Wr):
    bn = 1024
    return pl.pallas_call(
        _proj1_body,
        grid=(NP // bn,),
        in_specs=[
            pl.BlockSpec((bn, D), lambda i: (i, 0)),
            pl.BlockSpec((D, H1 * HID), lambda i: (0, 0)),
            pl.BlockSpec((D, H1 * HID), lambda i: (0, 0)),
        ],
        out_specs=(
            pl.BlockSpec((bn, H1 * HID), lambda i: (i, 0)),
            pl.BlockSpec((bn, H1 * HID), lambda i: (i, 0)),
        ),
        out_shape=(
            jax.ShapeDtypeStruct((NP, H1 * HID), jnp.float32),
            jax.ShapeDtypeStruct((NP, H1 * HID), jnp.float32),
        ),
    )(x, Wl, Wr)


def _proj2_body(q_ref, b_ref, wl_ref, wr_ref, ol_ref, or_ref):
    hcat = jnp.concatenate([q_ref[j] for j in range(H1)], axis=-1)
    hb = jnp.maximum(hcat + b_ref[...], 0.0)
    ol_ref[...] = hb @ wl_ref[...]
    or_ref[...] = hb @ wr_ref[...]


def _proj2(q1, b1, Wl2, Wr2):
    bn = 1024
    return pl.pallas_call(
        _proj2_body,
        grid=(NP // bn,),
        in_specs=[
            pl.BlockSpec((H1, bn, HID), lambda i: (0, i, 0)),
            pl.BlockSpec((1, H1 * HID), lambda i: (0, 0)),
            pl.BlockSpec((H1 * HID, HID), lambda i: (0, 0)),
            pl.BlockSpec((H1 * HID, HID), lambda i: (0, 0)),
        ],
        out_specs=(
            pl.BlockSpec((bn, HID), lambda i: (i, 0)),
            pl.BlockSpec((bn, HID), lambda i: (i, 0)),
        ),
        out_shape=(
            jax.ShapeDtypeStruct((NP, HID), jnp.float32),
            jax.ShapeDtypeStruct((NP, HID), jnp.float32),
        ),
    )(q1, b1.reshape(1, -1), Wl2, Wr2)


def _redsum_body(d_ref, o_ref):
    o_ref[...] = jnp.sum(d_ref[...], axis=0, keepdims=True)


def _reduce_den(den):
    hnp = den.shape[1]
    bn = 2048
    out = pl.pallas_call(
        _redsum_body,
        grid=(hnp // bn,),
        in_specs=[pl.BlockSpec((NW, bn), lambda i: (0, i))],
        out_specs=pl.BlockSpec((1, bn), lambda i: (0, i)),
        out_shape=jax.ShapeDtypeStruct((1, hnp), jnp.float32),
    )(den)
    return out.reshape(hnp)


def _tail_body(q_ref, b2_ref, batch_ref, doc_ref, Wdoc_ref, bdoc_ref,
               gamma_ref, beta_ref, Wfus_ref, bfus_ref, Wtask_ref, btask_ref,
               Wtime_ref, btime_ref, task_ref, time_ref):
    nblk = NP // 128
    iot = lax.broadcasted_iota(jnp.int32, (B, 128), 0)

    def blk(i, carry):
        pooled, cnt = carry
        hb = q_ref[0, pl.ds(i * 128, 128), :] + q_ref[1, pl.ds(i * 128, 128), :]
        hb = jnp.maximum(hb + b2_ref[...], 0.0)
        bv = batch_ref[i, :]
        onehot = (bv[None, :] == iot).astype(jnp.float32)
        pooled = pooled + onehot @ hb
        cnt = cnt + jnp.sum(onehot, axis=1, keepdims=True)
        return (pooled, cnt)

    pooled, cnt = lax.fori_loop(
        0, nblk, blk,
        (jnp.zeros((B, HID), jnp.float32), jnp.zeros((B, 1), jnp.float32)))
    pooled = pooled / jnp.maximum(cnt, 1.0)
    doc = jnp.maximum(doc_ref[...] @ Wdoc_ref[...] + bdoc_ref[...], 0.0)
    fusion = jnp.concatenate([pooled, doc], axis=1)
    mu = jnp.mean(fusion, axis=0, keepdims=True)
    var = jnp.mean((fusion - mu) ** 2, axis=0, keepdims=True)
    fusion = (fusion - mu) / jnp.sqrt(var + 1e-5) * gamma_ref[...] + beta_ref[...]
    fusion = jnp.maximum(fusion @ Wfus_ref[...] + bfus_ref[...], 0.0)
    task_ref[...] = fusion @ Wtask_ref[...] + btask_ref[...]
    time_ref[...] = fusion @ Wtime_ref[...] + btime_ref[...]


def _tail(q2, b2, batch2d, doc, Wdoc, bdoc, gamma, beta, Wfus, bfus,
          Wtask, btask, Wtime, btime):
    return pl.pallas_call(
        _tail_body,
        out_shape=(
            jax.ShapeDtypeStruct((B, OUT), jnp.float32),
            jax.ShapeDtypeStruct((B, 1), jnp.float32),
        ),
    )(q2, b2.reshape(1, -1), batch2d, doc, Wdoc, bdoc.reshape(1, -1),
      gamma.reshape(1, -1), beta.reshape(1, -1), Wfus, bfus.reshape(1, -1),
      Wtask, btask.reshape(1, -1), Wtime, btime.reshape(1, -1))


def kernel(x, edge_index, batch, doc_features, Wl1, Wr1, att1, b1, Wl2, Wr2,
           att2, b2, Wdoc, bdoc, gamma, beta, Wfus, bfus, Wtask, btask, Wtime,
           btime):
    loop = jnp.arange(N, dtype=edge_index.dtype)
    srcp = jnp.concatenate(
        [edge_index[0], loop, jnp.zeros((E_PAD - E_REAL,), jnp.int32)])
    dstp = jnp.concatenate(
        [edge_index[1], loop, jnp.zeros((E_PAD - E_REAL,), jnp.int32)])
    xp = jnp.pad(x, ((0, NP - N), (0, 0)))
    batchp = jnp.concatenate(
        [batch, jnp.full((NP - N,), B, jnp.int32)]).reshape(NP // 128, 128)

    xl1, xr1 = _proj1(xp, Wl1, Wr1)
    p1, den1p = _pass1_l1(xl1, xr1, srcp, dstp, att1.reshape(-1))
    den1 = _reduce_den(den1p)
    q1 = _pass2_l1(xl1.reshape(NP * H1, HID), srcp, dstp, p1, den1)

    xl2, xr2 = _proj2(q1, b1, Wl2, Wr2)
    p2, den2p = _pass1_l2(xl2, xr2, srcp, dstp, att2.reshape(-1))
    den2 = _reduce_den(den2p)
    q2 = _pass2_l2(xl2, srcp, dstp, p2, den2)

    task, time = _tail(q2, b2, batchp, doc_features, Wdoc, bdoc, gamma, beta,
                       Wfus, bfus, Wtask, btask, Wtime, btime)
    return (task, time)
